# baseline (speedup 1.0000x reference)
import functools

import jax
import jax.numpy as jnp
from jax import lax
from jax.experimental import pallas as pl
from jax.experimental.pallas import tpu as pltpu
from jax.experimental.pallas import tpu_sc as plsc

D = 128
S = 10
L = 16
NW = 32
C = 32
R = C * S
CHUNKS = 50
PER_TILE = C * CHUNKS
BPAD = PER_TILE * NW
GATHER_SPLITS = ((0, 128), (128, 128), (256, 64))


def _sc_mean(features, idx_flat):
    mesh = plsc.VectorSubcoreMesh(core_axis_name="c", subcore_axis_name="s")

    @functools.partial(
        pl.kernel,
        mesh=mesh,
        out_type=jax.ShapeDtypeStruct((BPAD, D), jnp.float32),
        scratch_types=[
            pltpu.VMEM((PER_TILE * S,), jnp.int32),
            pltpu.VMEM((R, D), jnp.float32),
            pltpu.VMEM((R, D), jnp.float32),
            pltpu.VMEM((C, D), jnp.float32),
            pltpu.VMEM((C, D), jnp.float32),
            pltpu.SemaphoreType.DMA,
            pltpu.SemaphoreType.DMA,
            pltpu.SemaphoreType.DMA,
            pltpu.SemaphoreType.DMA,
        ],
    )
    def k(feat_hbm, idx_hbm, out_hbm, idx_v, rows0, rows1, out0, out1,
          gsem0, gsem1, osem0, osem1):
        wid = lax.axis_index("s") * 2 + lax.axis_index("c")
        tile_node0 = wid * PER_TILE
        rows = (rows0, rows1)
        outs = (out0, out1)
        gsems = (gsem0, gsem1)
        osems = (osem0, osem1)

        def g_start(b, c):
            for g0, gn in GATHER_SPLITS:
                pltpu.async_copy(
                    feat_hbm.at[idx_v.at[pl.ds(c * R + g0, gn)]],
                    rows[b].at[pl.ds(g0, gn)],
                    gsems[b],
                )

        def g_wait(b, c):
            for g0, gn in GATHER_SPLITS:
                pltpu.make_async_copy(
                    feat_hbm.at[idx_v.at[pl.ds(c * R + g0, gn)]],
                    rows[b].at[pl.ds(g0, gn)],
                    gsems[b],
                ).wait()

        def o_start(b, c):
            pltpu.async_copy(
                outs[b], out_hbm.at[pl.ds(tile_node0 + c * C, C)], osems[b])

        def o_wait(b, c):
            pltpu.make_async_copy(
                outs[b], out_hbm.at[pl.ds(tile_node0 + c * C, C)],
                osems[b]).wait()

        def compute(b):
            rows_b = rows[b]
            out_b = outs[b]

            def node_body(n, carry):
                base = n * S
                for col in range(D // L):
                    acc = rows_b[base, pl.ds(col * L, L)]
                    for s_ in range(1, S):
                        acc = acc + rows_b[base + s_, pl.ds(col * L, L)]
                    out_b[n, pl.ds(col * L, L)] = acc * jnp.float32(0.1)
                return carry

            lax.fori_loop(0, C, node_body, 0)

        pltpu.sync_copy(idx_hbm.at[pl.ds(tile_node0 * S, PER_TILE * S)], idx_v)

        g_start(0, 0)
        g_start(1, 1)

        g_wait(0, 0)
        compute(0)
        o_start(0, 0)
        g_start(0, 2)

        g_wait(1, 1)
        compute(1)
        o_start(1, 1)
        g_start(1, 3)

        def steady(kk, carry):
            for b in (0, 1):
                c = 2 * kk + b
                g_wait(b, c)
                o_wait(b, c - 2)
                compute(b)
                o_start(b, c)
                g_start(b, c + 2)
            return carry

        lax.fori_loop(1, CHUNKS // 2 - 1, steady, 0)

        g_wait(0, CHUNKS - 2)
        o_wait(0, CHUNKS - 4)
        compute(0)
        o_start(0, CHUNKS - 2)

        g_wait(1, CHUNKS - 1)
        o_wait(1, CHUNKS - 3)
        compute(1)
        o_start(1, CHUNKS - 1)

        o_wait(0, CHUNKS - 2)
        o_wait(1, CHUNKS - 1)

    return k(features, idx_flat)


def kernel(features, nodes, to_neighs):
    b = to_neighs.shape[0]
    idx = to_neighs.astype(jnp.int32).reshape(-1)
    idx = jnp.pad(idx, (0, BPAD * S - idx.shape[0]))
    out = _sc_mean(features, idx)
    return out[:b]
